# baseline (device time: 59512 ns/iter reference)
import jax
import jax.numpy as jnp
from jax import lax
from jax.experimental import pallas as pl
from jax.experimental.pallas import tpu as pltpu

B, S, HL, D = 2, 1024, 16, 64
K = HL * D
N = 2048
NH = N // 2
S_HALF = S // 2

NQ = N // 4
OPS = (
    [(0, 0, 128, 0, NQ), (0, 0, 128, NQ, NQ), (0, 0, 128, NH, NH)]
    + [(0, r, 128, 0, N) for r in (128, 256, 384)]
    + [(1, r, 128, 0, N) for r in (0, 128, 256, 384)]
)
NSEND = len(OPS)


def kernel(O, Wo):
    OT = O.transpose(0, 2, 3, 1).reshape(B, K, S)

    def body(ot_hbm, w_hbm, out_ref, ot_vmem, wf_vmem, w16_vmem, acc_vmem,
             send_buf, recv_buf, send_sems, recv_sems, oload_sems,
             wload_sems):
        my_x = lax.axis_index("x")
        my_y = lax.axis_index("y")
        my_z = lax.axis_index("z")
        partner = 1 - my_x

        barrier_sem = pltpu.get_barrier_semaphore()
        pl.semaphore_signal(
            barrier_sem, inc=1,
            device_id=(partner, my_y, my_z),
            device_id_type=pl.DeviceIdType.MESH,
        )

        wloads = [
            pltpu.make_async_copy(
                w_hbm.at[:, q * NQ:(q + 1) * NQ],
                wf_vmem.at[:, q * NQ:(q + 1) * NQ],
                wload_sems.at[q],
            )
            for q in range(4)
        ]
        oloads = [
            pltpu.make_async_copy(
                ot_hbm.at[b], ot_vmem.at[b], oload_sems.at[b]
            )
            for b in range(B)
        ]
        wloads[0].start()
        oloads[0].start()
        for q in range(1, 4):
            wloads[q].start()
        oloads[1].start()

        def tdot(lhs, rhs):
            return lax.dot_general(
                lhs, rhs, (((0,), (0,)), ((), ())),
                preferred_element_type=jnp.float32,
            )

        w_ready = [False] * 4
        o_ready = [False, False]
        barrier_done = False

        rdmas = []
        for idx, (b, r0, rl, n0, nl) in enumerate(OPS):
            for q in range(4):
                if not w_ready[q] and n0 <= q * NQ < n0 + nl:
                    wloads[q].wait()
                    sl = slice(q * NQ, (q + 1) * NQ)
                    w16_vmem[:, sl] = wf_vmem[:, sl].astype(jnp.bfloat16)
                    w_ready[q] = True
            if not o_ready[b]:
                oloads[b].wait()
                o_ready[b] = True
            lhs = ot_vmem[
                b, :, pl.ds(partner * S_HALF + r0, rl)
            ].astype(jnp.bfloat16)
            rsl = slice(r0, r0 + rl)
            nsl = slice(n0, n0 + nl)
            send_buf[b, rsl, nsl] = tdot(lhs, w16_vmem[:, nsl]).astype(
                jnp.bfloat16
            )
            if not barrier_done:
                pl.semaphore_wait(barrier_sem, 1)
                barrier_done = True
            rdma = pltpu.make_async_remote_copy(
                src_ref=send_buf.at[b, rsl, nsl],
                dst_ref=recv_buf.at[b, rsl, nsl],
                send_sem=send_sems.at[idx],
                recv_sem=recv_sems.at[idx],
                device_id=(partner, my_y, my_z),
                device_id_type=pl.DeviceIdType.MESH,
            )
            rdma.start()
            rdmas.append(rdma)

        for b in range(B):
            lhs = ot_vmem[b, :, pl.ds(my_x * S_HALF, S_HALF)].astype(
                jnp.bfloat16
            )
            acc_vmem[b] = tdot(lhs, w16_vmem[...])

        for idx, (b, r0, rl, n0, nl) in enumerate(OPS):
            rdmas[idx].wait_recv()
            rsl = slice(r0, r0 + rl)
            nsl = slice(n0, n0 + nl)
            out_ref[b, rsl, nsl] = (
                acc_vmem[b, rsl, nsl]
                + recv_buf[b, rsl, nsl].astype(jnp.float32)
            ).astype(jnp.bfloat16)

        for rdma in rdmas:
            rdma.wait_send()

    return pl.pallas_call(
        body,
        out_shape=jax.ShapeDtypeStruct((B, S_HALF, N), jnp.bfloat16),
        in_specs=[
            pl.BlockSpec(memory_space=pl.ANY),
            pl.BlockSpec(memory_space=pl.ANY),
        ],
        out_specs=pl.BlockSpec(memory_space=pltpu.VMEM),
        scratch_shapes=[
            pltpu.VMEM((B, K, S), jnp.float32),
            pltpu.VMEM((K, N), jnp.float32),
            pltpu.VMEM((K, N), jnp.bfloat16),
            pltpu.VMEM((B, S_HALF, N), jnp.float32),
            pltpu.VMEM((B, S_HALF, N), jnp.bfloat16),
            pltpu.VMEM((B, S_HALF, N), jnp.bfloat16),
            pltpu.SemaphoreType.DMA((NSEND,)),
            pltpu.SemaphoreType.DMA((NSEND,)),
            pltpu.SemaphoreType.DMA((B,)),
            pltpu.SemaphoreType.DMA((4,)),
        ],
        compiler_params=pltpu.CompilerParams(
            collective_id=0,
            vmem_limit_bytes=100 * 1024 * 1024,
        ),
    )(OT, Wo)


# device time: 58640 ns/iter; 1.0149x vs baseline; 1.0149x over previous
import jax
import jax.numpy as jnp
from jax import lax
from jax.experimental import pallas as pl
from jax.experimental.pallas import tpu as pltpu

B, S, HL, D = 2, 1024, 16, 64
K = HL * D
N = 2048
NH = N // 2
S_HALF = S // 2

NQ = N // 4
OPS = (
    [(0, 0, 128, 0, NQ), (0, 0, 128, NQ, NQ), (0, 0, 128, NH, NH)]
    + [(0, r, 128, 0, N) for r in (128, 256, 384)]
    + [(1, r, 128, 0, N) for r in (0, 128, 256, 384)]
)
NSEND = len(OPS)


def kernel(O, Wo):
    OT = O.transpose(0, 2, 3, 1).reshape(B, K, S)

    def body(ot_hbm, w_hbm, out_hbm, ot_vmem, wf_vmem, w16_vmem, acc_vmem,
             out_vmem, send_buf, recv_buf, send_sems, recv_sems, oload_sems,
             wload_sems, store_sems):
        my_x = lax.axis_index("x")
        my_y = lax.axis_index("y")
        my_z = lax.axis_index("z")
        partner = 1 - my_x

        barrier_sem = pltpu.get_barrier_semaphore()
        pl.semaphore_signal(
            barrier_sem, inc=1,
            device_id=(partner, my_y, my_z),
            device_id_type=pl.DeviceIdType.MESH,
        )

        wloads = [
            pltpu.make_async_copy(
                w_hbm.at[:, q * NQ:(q + 1) * NQ],
                wf_vmem.at[:, q * NQ:(q + 1) * NQ],
                wload_sems.at[q],
            )
            for q in range(4)
        ]
        oloads = [
            pltpu.make_async_copy(
                ot_hbm.at[b], ot_vmem.at[b], oload_sems.at[b]
            )
            for b in range(B)
        ]
        wloads[0].start()
        oloads[0].start()
        for q in range(1, 4):
            wloads[q].start()
        oloads[1].start()

        def tdot(lhs, rhs):
            return lax.dot_general(
                lhs, rhs, (((0,), (0,)), ((), ())),
                preferred_element_type=jnp.float32,
            )

        w_ready = [False] * 4
        o_ready = [False, False]
        barrier_done = False

        rdmas = []
        for idx, (b, r0, rl, n0, nl) in enumerate(OPS):
            for q in range(4):
                if not w_ready[q] and n0 <= q * NQ < n0 + nl:
                    wloads[q].wait()
                    sl = slice(q * NQ, (q + 1) * NQ)
                    w16_vmem[:, sl] = wf_vmem[:, sl].astype(jnp.bfloat16)
                    w_ready[q] = True
            if not o_ready[b]:
                oloads[b].wait()
                o_ready[b] = True
            lhs = ot_vmem[
                b, :, pl.ds(partner * S_HALF + r0, rl)
            ].astype(jnp.bfloat16)
            rsl = slice(r0, r0 + rl)
            nsl = slice(n0, n0 + nl)
            send_buf[b, rsl, nsl] = tdot(lhs, w16_vmem[:, nsl]).astype(
                jnp.bfloat16
            )
            if not barrier_done:
                pl.semaphore_wait(barrier_sem, 1)
                barrier_done = True
            rdma = pltpu.make_async_remote_copy(
                src_ref=send_buf.at[b, rsl, nsl],
                dst_ref=recv_buf.at[b, rsl, nsl],
                send_sem=send_sems.at[idx],
                recv_sem=recv_sems.at[idx],
                device_id=(partner, my_y, my_z),
                device_id_type=pl.DeviceIdType.MESH,
            )
            rdma.start()
            rdmas.append(rdma)

        for b in range(B):
            lhs = ot_vmem[b, :, pl.ds(my_x * S_HALF, S_HALF)].astype(
                jnp.bfloat16
            )
            acc_vmem[b] = tdot(lhs, w16_vmem[...])

        stores = []
        for idx, (b, r0, rl, n0, nl) in enumerate(OPS):
            rdmas[idx].wait_recv()
            rsl = slice(r0, r0 + rl)
            nsl = slice(n0, n0 + nl)
            out_vmem[b, rsl, nsl] = (
                acc_vmem[b, rsl, nsl]
                + recv_buf[b, rsl, nsl].astype(jnp.float32)
            ).astype(jnp.bfloat16)
            store = pltpu.make_async_copy(
                out_vmem.at[b, rsl, nsl], out_hbm.at[b, rsl, nsl],
                store_sems.at[idx],
            )
            store.start()
            stores.append(store)

        for rdma in rdmas:
            rdma.wait_send()
        for store in stores:
            store.wait()

    return pl.pallas_call(
        body,
        out_shape=jax.ShapeDtypeStruct((B, S_HALF, N), jnp.bfloat16),
        in_specs=[
            pl.BlockSpec(memory_space=pl.ANY),
            pl.BlockSpec(memory_space=pl.ANY),
        ],
        out_specs=pl.BlockSpec(memory_space=pl.ANY),
        scratch_shapes=[
            pltpu.VMEM((B, K, S), jnp.float32),
            pltpu.VMEM((K, N), jnp.float32),
            pltpu.VMEM((K, N), jnp.bfloat16),
            pltpu.VMEM((B, S_HALF, N), jnp.float32),
            pltpu.VMEM((B, S_HALF, N), jnp.bfloat16),
            pltpu.VMEM((B, S_HALF, N), jnp.bfloat16),
            pltpu.VMEM((B, S_HALF, N), jnp.bfloat16),
            pltpu.SemaphoreType.DMA((NSEND,)),
            pltpu.SemaphoreType.DMA((NSEND,)),
            pltpu.SemaphoreType.DMA((B,)),
            pltpu.SemaphoreType.DMA((4,)),
            pltpu.SemaphoreType.DMA((NSEND,)),
        ],
        compiler_params=pltpu.CompilerParams(
            collective_id=0,
            vmem_limit_bytes=100 * 1024 * 1024,
        ),
    )(OT, Wo)
